# Initial kernel scaffold; baseline (speedup 1.0000x reference)
import jax
import jax.numpy as jnp
from jax.experimental import pallas as pl


def kernel(x, edge_index, W1, b1, W2, b2, W3, b3):
    raise NotImplementedError("write your pallas kernel here")


# speedup vs baseline: 6.3671x; 6.3671x over previous
import functools

import jax
import jax.numpy as jnp
from jax import lax
from jax.experimental import pallas as pl
from jax.experimental.pallas import tpu as pltpu
from jax.experimental.pallas import tpu_sc as plsc

N = 10000
D = 128
DH = 64
NP = 10240
PAD_ROW = 10000

NC = 2
NS = 16
NW = NC * NS
EB = 128
S = 80
EP = NW * S * EB

RPT = NP // NS
BR = 640
GRID = NP // BR

_mesh = plsc.VectorSubcoreMesh(core_axis_name="c", subcore_axis_name="s")


@functools.partial(
    pl.kernel,
    out_type=jax.ShapeDtypeStruct((NC, NP, 16), jnp.float32),
    mesh=_mesh,
    scratch_types=[
        pltpu.VMEM((S, EB), jnp.int32),
        pltpu.VMEM((EB, 16), jnp.float32),
        pltpu.VMEM_SHARED((NP, 16), jnp.float32),
    ],
    compiler_params=pltpu.CompilerParams(use_tc_tiling_on_sc=False),
)
def _deg_kernel(dst_hbm, zeros_hbm, ones_hbm, out_hbm, dst_v, ones_v, acc_sh):
    c = lax.axis_index("c")
    s = lax.axis_index("s")
    wid = c * NS + s
    base = s * RPT
    pltpu.sync_copy(zeros_hbm.at[pl.ds(base, RPT)], acc_sh.at[pl.ds(base, RPT)])
    pltpu.sync_copy(ones_hbm, ones_v)
    pltpu.sync_copy(dst_hbm.at[wid], dst_v)
    plsc.subcore_barrier()

    def body(j, carry):
        pltpu.sync_copy(ones_v, acc_sh.at[dst_v.at[j]], add=True)
        return carry

    lax.fori_loop(0, S, body, 0)
    plsc.subcore_barrier()
    pltpu.sync_copy(acc_sh.at[pl.ds(base, RPT)], out_hbm.at[c, pl.ds(base, RPT)])


@functools.partial(
    pl.kernel,
    out_type=jax.ShapeDtypeStruct((NC, NP, DH), jnp.float32),
    mesh=_mesh,
    scratch_types=[
        pltpu.VMEM((S, EB), jnp.int32),
        pltpu.VMEM((S, EB), jnp.int32),
        pltpu.VMEM((EB, DH), jnp.float32),
        pltpu.VMEM((EB, DH), jnp.float32),
        pltpu.VMEM_SHARED((NP, DH), jnp.float32),
        pltpu.SemaphoreType.DMA,
        pltpu.SemaphoreType.DMA,
    ],
    compiler_params=pltpu.CompilerParams(use_tc_tiling_on_sc=False),
)
def _scatter_kernel(g_hbm, src_hbm, dst_hbm, zeros_hbm, out_hbm,
                    src_v, dst_v, rows_a, rows_b, acc_sh, sem_a, sem_b):
    c = lax.axis_index("c")
    s = lax.axis_index("s")
    wid = c * NS + s
    base = s * RPT
    pltpu.sync_copy(zeros_hbm.at[pl.ds(base, RPT)], acc_sh.at[pl.ds(base, RPT)])
    pltpu.sync_copy(src_hbm.at[wid], src_v)
    pltpu.sync_copy(dst_hbm.at[wid], dst_v)
    plsc.subcore_barrier()

    pltpu.async_copy(g_hbm.at[src_v.at[0]], rows_a, sem_a)

    def body(jj, carry):
        j = jj * 2
        pltpu.make_async_copy(g_hbm.at[src_v.at[j]], rows_a, sem_a).wait()
        pltpu.async_copy(g_hbm.at[src_v.at[j + 1]], rows_b, sem_b)
        pltpu.sync_copy(rows_a, acc_sh.at[dst_v.at[j]], add=True)
        pltpu.make_async_copy(g_hbm.at[src_v.at[j + 1]], rows_b, sem_b).wait()

        @pl.when(jj < S // 2 - 1)
        def _():
            pltpu.async_copy(g_hbm.at[src_v.at[j + 2]], rows_a, sem_a)

        pltpu.sync_copy(rows_b, acc_sh.at[dst_v.at[j + 1]], add=True)
        return carry

    lax.fori_loop(0, S // 2, body, 0)
    plsc.subcore_barrier()
    pltpu.sync_copy(acc_sh.at[pl.ds(base, RPT)], out_hbm.at[c, pl.ds(base, RPT)])


def _first_body(deg_ref, x_ref, w_ref, glo_ref, ghi_ref, dinv_ref):
    deg = deg_ref[0, :, 0] + deg_ref[1, :, 0] + 1.0
    dinv = lax.rsqrt(deg)[:, None]
    dinv_ref[...] = dinv
    g = jnp.dot(x_ref[...], w_ref[...],
                preferred_element_type=jnp.float32) * dinv
    glo_ref[...] = g[:, :DH]
    ghi_ref[...] = g[:, DH:]


_first_tc = pl.pallas_call(
    _first_body,
    grid=(GRID,),
    in_specs=[
        pl.BlockSpec((NC, BR, 16), lambda i: (0, i, 0)),
        pl.BlockSpec((BR, D), lambda i: (i, 0)),
        pl.BlockSpec((D, D), lambda i: (0, 0)),
    ],
    out_specs=[
        pl.BlockSpec((BR, DH), lambda i: (i, 0)),
        pl.BlockSpec((BR, DH), lambda i: (i, 0)),
        pl.BlockSpec((BR, 1), lambda i: (i, 0)),
    ],
    out_shape=[
        jax.ShapeDtypeStruct((NP, DH), jnp.float32),
        jax.ShapeDtypeStruct((NP, DH), jnp.float32),
        jax.ShapeDtypeStruct((NP, 1), jnp.float32),
    ],
)


def _mid_body(alo_ref, ahi_ref, glo_ref, ghi_ref, dinv_ref, b_ref, w_ref,
              gnlo_ref, gnhi_ref):
    dinv = dinv_ref[...]
    tlo = dinv * (alo_ref[0] + alo_ref[1] + glo_ref[...]) + b_ref[:, :DH]
    thi = dinv * (ahi_ref[0] + ahi_ref[1] + ghi_ref[...]) + b_ref[:, DH:]
    t = jnp.concatenate([tlo, thi], axis=1)
    gn = jnp.dot(t, w_ref[...], preferred_element_type=jnp.float32) * dinv
    gnlo_ref[...] = gn[:, :DH]
    gnhi_ref[...] = gn[:, DH:]


_mid_tc = pl.pallas_call(
    _mid_body,
    grid=(GRID,),
    in_specs=[
        pl.BlockSpec((NC, BR, DH), lambda i: (0, i, 0)),
        pl.BlockSpec((NC, BR, DH), lambda i: (0, i, 0)),
        pl.BlockSpec((BR, DH), lambda i: (i, 0)),
        pl.BlockSpec((BR, DH), lambda i: (i, 0)),
        pl.BlockSpec((BR, 1), lambda i: (i, 0)),
        pl.BlockSpec((1, D), lambda i: (0, 0)),
        pl.BlockSpec((D, D), lambda i: (0, 0)),
    ],
    out_specs=[
        pl.BlockSpec((BR, DH), lambda i: (i, 0)),
        pl.BlockSpec((BR, DH), lambda i: (i, 0)),
    ],
    out_shape=[
        jax.ShapeDtypeStruct((NP, DH), jnp.float32),
        jax.ShapeDtypeStruct((NP, DH), jnp.float32),
    ],
)


def _final_body(alo_ref, ahi_ref, glo_ref, ghi_ref, dinv_ref, b_ref, out_ref):
    dinv = dinv_ref[...]
    tlo = dinv * (alo_ref[0] + alo_ref[1] + glo_ref[...]) + b_ref[:, :DH]
    thi = dinv * (ahi_ref[0] + ahi_ref[1] + ghi_ref[...]) + b_ref[:, DH:]
    out_ref[...] = jnp.concatenate([tlo, thi], axis=1)


_final_tc = pl.pallas_call(
    _final_body,
    grid=(GRID,),
    in_specs=[
        pl.BlockSpec((NC, BR, DH), lambda i: (0, i, 0)),
        pl.BlockSpec((NC, BR, DH), lambda i: (0, i, 0)),
        pl.BlockSpec((BR, DH), lambda i: (i, 0)),
        pl.BlockSpec((BR, DH), lambda i: (i, 0)),
        pl.BlockSpec((BR, 1), lambda i: (i, 0)),
        pl.BlockSpec((1, D), lambda i: (0, 0)),
    ],
    out_specs=pl.BlockSpec((BR, D), lambda i: (i, 0)),
    out_shape=jax.ShapeDtypeStruct((NP, D), jnp.float32),
)


def kernel(x, edge_index, W1, b1, W2, b2, W3, b3):
    E = edge_index.shape[1]
    pad = EP - E
    src = jnp.concatenate(
        [edge_index[0].astype(jnp.int32),
         jnp.full((pad,), PAD_ROW, jnp.int32)]).reshape(NW, S, EB)
    dst = jnp.concatenate(
        [edge_index[1].astype(jnp.int32),
         jnp.full((pad,), PAD_ROW, jnp.int32)]).reshape(NW, S, EB)
    x_p = jnp.concatenate([x, jnp.zeros((NP - N, D), jnp.float32)])
    zeros16 = jnp.zeros((NP, 16), jnp.float32)
    ones16 = jnp.ones((EB, 16), jnp.float32)
    zerosH = jnp.zeros((NP, DH), jnp.float32)
    b1r = b1.reshape(1, D)
    b2r = b2.reshape(1, D)
    b3r = b3.reshape(1, D)

    deg_parts = _deg_kernel(dst, zeros16, ones16)
    glo, ghi, dinv = _first_tc(deg_parts, x_p, W1)
    for (b_r, W_next) in ((b1r, W2), (b2r, W3)):
        alo = _scatter_kernel(glo, src, dst, zerosH)
        ahi = _scatter_kernel(ghi, src, dst, zerosH)
        glo, ghi = _mid_tc(alo, ahi, glo, ghi, dinv, b_r, W_next)
    alo = _scatter_kernel(glo, src, dst, zerosH)
    ahi = _scatter_kernel(ghi, src, dst, zerosH)
    out = _final_tc(alo, ahi, glo, ghi, dinv, b3r)
    return out[:N]


# speedup vs baseline: 16.5827x; 2.6044x over previous
import functools

import jax
import jax.numpy as jnp
from jax import lax
from jax.experimental import pallas as pl
from jax.experimental.pallas import tpu as pltpu
from jax.experimental.pallas import tpu_sc as plsc

N = 10000
D = 128
DH = 64
NP = 10240
PAD_ROW = 10000

NC = 2
NS = 16
NW = NC * NS
EB = 128
S = 80
EP = NW * S * EB

RPT = NP // NS
BR = 640
GRID = NP // BR

_mesh = plsc.VectorSubcoreMesh(core_axis_name="c", subcore_axis_name="s")


@functools.partial(
    pl.kernel,
    out_type=jax.ShapeDtypeStruct((NC, NP, 16), jnp.float32),
    mesh=_mesh,
    scratch_types=[
        pltpu.VMEM((S, EB), jnp.int32),
        pltpu.VMEM((EB, 16), jnp.float32),
        pltpu.VMEM_SHARED((NP, 16), jnp.float32),
    ],
    compiler_params=pltpu.CompilerParams(use_tc_tiling_on_sc=False),
)
def _deg_kernel(dst_hbm, zeros_hbm, ones_hbm, out_hbm, dst_v, ones_v, acc_sh):
    c = lax.axis_index("c")
    s = lax.axis_index("s")
    wid = c * NS + s
    base = s * RPT
    pltpu.sync_copy(zeros_hbm.at[pl.ds(base, RPT)], acc_sh.at[pl.ds(base, RPT)])
    pltpu.sync_copy(ones_hbm, ones_v)
    pltpu.sync_copy(dst_hbm.at[wid], dst_v)
    plsc.subcore_barrier()

    def body(j, carry):
        pltpu.sync_copy(ones_v, acc_sh.at[dst_v.at[j]], add=True)
        return carry

    lax.fori_loop(0, S, body, 0)
    plsc.subcore_barrier()
    pltpu.sync_copy(acc_sh.at[pl.ds(base, RPT)], out_hbm.at[c, pl.ds(base, RPT)])


@functools.partial(
    pl.kernel,
    out_type=jax.ShapeDtypeStruct((NC, NP, DH), jnp.float32),
    mesh=_mesh,
    scratch_types=[
        pltpu.VMEM((S, EB), jnp.int32),
        pltpu.VMEM((S, EB), jnp.int32),
        pltpu.VMEM((EB, DH), jnp.float32),
        pltpu.VMEM((EB, DH), jnp.float32),
        pltpu.VMEM_SHARED((NP, DH), jnp.float32),
        pltpu.SemaphoreType.DMA,
        pltpu.SemaphoreType.DMA,
    ],
    compiler_params=pltpu.CompilerParams(use_tc_tiling_on_sc=False),
)
def _scatter_kernel(g_hbm, src_hbm, dst_hbm, zeros_hbm, out_hbm,
                    src_v, dst_v, rows_a, rows_b, acc_sh, sem_a, sem_b):
    c = lax.axis_index("c")
    s = lax.axis_index("s")
    wid = c * NS + s
    base = s * RPT
    pltpu.sync_copy(zeros_hbm.at[pl.ds(base, RPT)], acc_sh.at[pl.ds(base, RPT)])
    pltpu.sync_copy(src_hbm.at[wid], src_v)
    pltpu.sync_copy(dst_hbm.at[wid], dst_v)
    plsc.subcore_barrier()

    pltpu.async_copy(g_hbm.at[src_v.at[0]], rows_a, sem_a)

    def body(jj, carry):
        j = jj * 2
        pltpu.make_async_copy(g_hbm.at[src_v.at[j]], rows_a, sem_a).wait()
        pltpu.async_copy(g_hbm.at[src_v.at[j + 1]], rows_b, sem_b)
        pltpu.sync_copy(rows_a, acc_sh.at[dst_v.at[j]], add=True)
        pltpu.make_async_copy(g_hbm.at[src_v.at[j + 1]], rows_b, sem_b).wait()

        @pl.when(jj < S // 2 - 1)
        def _():
            pltpu.async_copy(g_hbm.at[src_v.at[j + 2]], rows_a, sem_a)

        pltpu.sync_copy(rows_b, acc_sh.at[dst_v.at[j + 1]], add=True)
        return carry

    lax.fori_loop(0, S // 2, body, 0)
    plsc.subcore_barrier()
    pltpu.sync_copy(acc_sh.at[pl.ds(base, RPT)], out_hbm.at[c, pl.ds(base, RPT)])


def _first_body(deg_ref, x_ref, w_ref, glo_ref, ghi_ref, dinv_ref):
    deg = deg_ref[0, :, 0] + deg_ref[1, :, 0] + 1.0
    dinv = lax.rsqrt(deg)[:, None]
    dinv_ref[...] = dinv
    g = jnp.dot(x_ref[...], w_ref[...],
                preferred_element_type=jnp.float32) * dinv
    glo_ref[...] = g[:, :DH]
    ghi_ref[...] = g[:, DH:]


_first_tc = pl.pallas_call(
    _first_body,
    grid=(GRID,),
    in_specs=[
        pl.BlockSpec((NC, BR, 16), lambda i: (0, i, 0)),
        pl.BlockSpec((BR, D), lambda i: (i, 0)),
        pl.BlockSpec((D, D), lambda i: (0, 0)),
    ],
    out_specs=[
        pl.BlockSpec((BR, DH), lambda i: (i, 0)),
        pl.BlockSpec((BR, DH), lambda i: (i, 0)),
        pl.BlockSpec((BR, 1), lambda i: (i, 0)),
    ],
    out_shape=[
        jax.ShapeDtypeStruct((NP, DH), jnp.float32),
        jax.ShapeDtypeStruct((NP, DH), jnp.float32),
        jax.ShapeDtypeStruct((NP, 1), jnp.float32),
    ],
)


def _mid_body(alo_ref, ahi_ref, glo_ref, ghi_ref, dinv_ref, b_ref, w_ref,
              gnlo_ref, gnhi_ref):
    dinv = dinv_ref[...]
    tlo = dinv * (alo_ref[0] + alo_ref[1] + glo_ref[...]) + b_ref[:, :DH]
    thi = dinv * (ahi_ref[0] + ahi_ref[1] + ghi_ref[...]) + b_ref[:, DH:]
    t = jnp.concatenate([tlo, thi], axis=1)
    gn = jnp.dot(t, w_ref[...], preferred_element_type=jnp.float32) * dinv
    gnlo_ref[...] = gn[:, :DH]
    gnhi_ref[...] = gn[:, DH:]


_mid_tc = pl.pallas_call(
    _mid_body,
    grid=(GRID,),
    in_specs=[
        pl.BlockSpec((NC, BR, DH), lambda i: (0, i, 0)),
        pl.BlockSpec((NC, BR, DH), lambda i: (0, i, 0)),
        pl.BlockSpec((BR, DH), lambda i: (i, 0)),
        pl.BlockSpec((BR, DH), lambda i: (i, 0)),
        pl.BlockSpec((BR, 1), lambda i: (i, 0)),
        pl.BlockSpec((1, D), lambda i: (0, 0)),
        pl.BlockSpec((D, D), lambda i: (0, 0)),
    ],
    out_specs=[
        pl.BlockSpec((BR, DH), lambda i: (i, 0)),
        pl.BlockSpec((BR, DH), lambda i: (i, 0)),
    ],
    out_shape=[
        jax.ShapeDtypeStruct((NP, DH), jnp.float32),
        jax.ShapeDtypeStruct((NP, DH), jnp.float32),
    ],
)


def _final_body(alo_ref, ahi_ref, glo_ref, ghi_ref, dinv_ref, b_ref, out_ref):
    dinv = dinv_ref[...]
    tlo = dinv * (alo_ref[0] + alo_ref[1] + glo_ref[...]) + b_ref[:, :DH]
    thi = dinv * (ahi_ref[0] + ahi_ref[1] + ghi_ref[...]) + b_ref[:, DH:]
    out_ref[...] = jnp.concatenate([tlo, thi], axis=1)


_final_tc = pl.pallas_call(
    _final_body,
    grid=(GRID,),
    in_specs=[
        pl.BlockSpec((NC, BR, DH), lambda i: (0, i, 0)),
        pl.BlockSpec((NC, BR, DH), lambda i: (0, i, 0)),
        pl.BlockSpec((BR, DH), lambda i: (i, 0)),
        pl.BlockSpec((BR, DH), lambda i: (i, 0)),
        pl.BlockSpec((BR, 1), lambda i: (i, 0)),
        pl.BlockSpec((1, D), lambda i: (0, 0)),
    ],
    out_specs=pl.BlockSpec((BR, D), lambda i: (i, 0)),
    out_shape=jax.ShapeDtypeStruct((NP, D), jnp.float32),
)


def kernel(x, edge_index, W1, b1, W2, b2, W3, b3):
    E = edge_index.shape[1]
    pad = EP - E
    pad_idx = PAD_ROW + (jnp.arange(pad, dtype=jnp.int32) % (NP - PAD_ROW))
    src = jnp.concatenate(
        [edge_index[0].astype(jnp.int32), pad_idx]).reshape(NW, S, EB)
    dst = jnp.concatenate(
        [edge_index[1].astype(jnp.int32), pad_idx]).reshape(NW, S, EB)
    x_p = jnp.concatenate([x, jnp.zeros((NP - N, D), jnp.float32)])
    zeros16 = jnp.zeros((NP, 16), jnp.float32)
    ones16 = jnp.ones((EB, 16), jnp.float32)
    zerosH = jnp.zeros((NP, DH), jnp.float32)
    b1r = b1.reshape(1, D)
    b2r = b2.reshape(1, D)
    b3r = b3.reshape(1, D)

    deg_parts = _deg_kernel(dst, zeros16, ones16)
    glo, ghi, dinv = _first_tc(deg_parts, x_p, W1)
    for (b_r, W_next) in ((b1r, W2), (b2r, W3)):
        alo = _scatter_kernel(glo, src, dst, zerosH)
        ahi = _scatter_kernel(ghi, src, dst, zerosH)
        glo, ghi = _mid_tc(alo, ahi, glo, ghi, dinv, b_r, W_next)
    alo = _scatter_kernel(glo, src, dst, zerosH)
    ahi = _scatter_kernel(ghi, src, dst, zerosH)
    out = _final_tc(alo, ahi, glo, ghi, dinv, b3r)
    return out[:N]


# speedup vs baseline: 22.6978x; 1.3688x over previous
import functools

import jax
import jax.numpy as jnp
from jax import lax
from jax.experimental import pallas as pl
from jax.experimental.pallas import tpu as pltpu
from jax.experimental.pallas import tpu_sc as plsc

N = 10000
D = 128
NP = 10240
PAD_ROW = 10000

NC = 2
NS = 16
NW = NC * NS
EB = 128
S = 80
EP = NW * S * EB

RPT = NP // NS
BR = 640
GRID = NP // BR

_mesh = plsc.VectorSubcoreMesh(core_axis_name="c", subcore_axis_name="s")
_sc_params = pltpu.CompilerParams(use_tc_tiling_on_sc=False)


@functools.partial(
    pl.kernel,
    out_type=jax.ShapeDtypeStruct((NC, NP, 16), jnp.float32),
    mesh=_mesh,
    scratch_types=[
        pltpu.VMEM((S, EB), jnp.int32),
        pltpu.VMEM((EB, 16), jnp.float32),
        pltpu.VMEM_SHARED((NP, 16), jnp.float32),
    ],
    compiler_params=_sc_params,
)
def _deg_kernel(dst_hbm, zeros_hbm, ones_hbm, out_hbm, dst_v, ones_v, acc_sh):
    c = lax.axis_index("c")
    s = lax.axis_index("s")
    wid = c * NS + s
    base = s * RPT
    pltpu.sync_copy(zeros_hbm.at[pl.ds(base, RPT)], acc_sh.at[pl.ds(base, RPT)])
    pltpu.sync_copy(ones_hbm, ones_v)
    pltpu.sync_copy(dst_hbm.at[wid], dst_v)
    plsc.subcore_barrier()

    def body(j, carry):
        pltpu.sync_copy(ones_v, acc_sh.at[dst_v.at[j]], add=True)
        return carry

    lax.fori_loop(0, S, body, 0)
    plsc.subcore_barrier()
    pltpu.sync_copy(acc_sh.at[pl.ds(base, RPT)], out_hbm.at[c, pl.ds(base, RPT)])


CH = 16
NCH = S // CH


@functools.partial(
    pl.kernel,
    out_type=jax.ShapeDtypeStruct((NC, NP, D), jnp.float32),
    mesh=_mesh,
    scratch_types=[
        pltpu.VMEM((CH, EB), jnp.int32),
        pltpu.VMEM((CH, EB), jnp.int32),
        pltpu.VMEM((CH, EB), jnp.int32),
        pltpu.VMEM((CH, EB), jnp.int32),
        pltpu.VMEM((EB, D), jnp.float32),
        pltpu.VMEM((EB, D), jnp.float32),
        pltpu.VMEM_SHARED((NP, D), jnp.float32),
        pltpu.SemaphoreType.DMA,
        pltpu.SemaphoreType.DMA,
        pltpu.SemaphoreType.DMA,
    ],
    compiler_params=_sc_params,
)
def _scatter_kernel(g_hbm, src_hbm, dst_hbm, zeros_hbm, out_hbm,
                    src_a, dst_a, src_b, dst_b, rows_a, rows_b, acc_sh,
                    sem_a, sem_b, sem_i):
    c = lax.axis_index("c")
    s = lax.axis_index("s")
    wid = c * NS + s
    base = s * RPT
    pltpu.sync_copy(zeros_hbm.at[pl.ds(base, RPT)], acc_sh.at[pl.ds(base, RPT)])
    pltpu.async_copy(src_hbm.at[wid, pl.ds(0, CH)], src_a, sem_i)
    pltpu.async_copy(dst_hbm.at[wid, pl.ds(0, CH)], dst_a, sem_i)
    pltpu.make_async_copy(src_hbm.at[wid, pl.ds(0, CH)], src_a, sem_i).wait()
    pltpu.make_async_copy(dst_hbm.at[wid, pl.ds(0, CH)], dst_a, sem_i).wait()
    plsc.subcore_barrier()

    for ch in range(NCH):
        src_v, dst_v = (src_a, dst_a) if ch % 2 == 0 else (src_b, dst_b)
        src_n, dst_n = (src_b, dst_b) if ch % 2 == 0 else (src_a, dst_a)
        if ch + 1 < NCH:
            pltpu.async_copy(src_hbm.at[wid, pl.ds((ch + 1) * CH, CH)],
                             src_n, sem_i)
            pltpu.async_copy(dst_hbm.at[wid, pl.ds((ch + 1) * CH, CH)],
                             dst_n, sem_i)
        pltpu.async_copy(g_hbm.at[src_v.at[0]], rows_a, sem_a)

        def body(jj, carry, src_v=src_v, dst_v=dst_v):
            j = jj * 2
            pltpu.make_async_copy(g_hbm.at[src_v.at[j]], rows_a, sem_a).wait()
            pltpu.async_copy(g_hbm.at[src_v.at[j + 1]], rows_b, sem_b)
            pltpu.sync_copy(rows_a, acc_sh.at[dst_v.at[j]], add=True)
            pltpu.make_async_copy(g_hbm.at[src_v.at[j + 1]], rows_b, sem_b).wait()

            @pl.when(jj < CH // 2 - 1)
            def _():
                pltpu.async_copy(g_hbm.at[src_v.at[j + 2]], rows_a, sem_a)

            pltpu.sync_copy(rows_b, acc_sh.at[dst_v.at[j + 1]], add=True)
            return carry

        lax.fori_loop(0, CH // 2, body, 0)
        if ch + 1 < NCH:
            pltpu.make_async_copy(src_hbm.at[wid, pl.ds((ch + 1) * CH, CH)],
                                  src_n, sem_i).wait()
            pltpu.make_async_copy(dst_hbm.at[wid, pl.ds((ch + 1) * CH, CH)],
                                  dst_n, sem_i).wait()
    plsc.subcore_barrier()
    pltpu.sync_copy(acc_sh.at[pl.ds(base, RPT)], out_hbm.at[c, pl.ds(base, RPT)])


def _first_body(deg_ref, x_ref, w_ref, g_ref, dinv_ref):
    deg = deg_ref[0, :, 0] + deg_ref[1, :, 0] + 1.0
    dinv = lax.rsqrt(deg)[:, None]
    dinv_ref[...] = dinv
    g_ref[...] = jnp.dot(x_ref[...], w_ref[...],
                         preferred_element_type=jnp.float32) * dinv


_first_tc = pl.pallas_call(
    _first_body,
    grid=(GRID,),
    in_specs=[
        pl.BlockSpec((NC, BR, 16), lambda i: (0, i, 0)),
        pl.BlockSpec((BR, D), lambda i: (i, 0)),
        pl.BlockSpec((D, D), lambda i: (0, 0)),
    ],
    out_specs=[
        pl.BlockSpec((BR, D), lambda i: (i, 0)),
        pl.BlockSpec((BR, 1), lambda i: (i, 0)),
    ],
    out_shape=[
        jax.ShapeDtypeStruct((NP, D), jnp.float32),
        jax.ShapeDtypeStruct((NP, 1), jnp.float32),
    ],
)


def _mid_body(acc_ref, g_ref, dinv_ref, b_ref, w_ref, gn_ref):
    dinv = dinv_ref[...]
    t = dinv * (acc_ref[0] + acc_ref[1] + g_ref[...]) + b_ref[...]
    gn_ref[...] = jnp.dot(t, w_ref[...],
                          preferred_element_type=jnp.float32) * dinv


_mid_tc = pl.pallas_call(
    _mid_body,
    grid=(GRID,),
    in_specs=[
        pl.BlockSpec((NC, BR, D), lambda i: (0, i, 0)),
        pl.BlockSpec((BR, D), lambda i: (i, 0)),
        pl.BlockSpec((BR, 1), lambda i: (i, 0)),
        pl.BlockSpec((1, D), lambda i: (0, 0)),
        pl.BlockSpec((D, D), lambda i: (0, 0)),
    ],
    out_specs=pl.BlockSpec((BR, D), lambda i: (i, 0)),
    out_shape=jax.ShapeDtypeStruct((NP, D), jnp.float32),
)


def _final_body(acc_ref, g_ref, dinv_ref, b_ref, out_ref):
    out_ref[...] = (dinv_ref[...] * (acc_ref[0] + acc_ref[1] + g_ref[...])
                    + b_ref[...])


_final_tc = pl.pallas_call(
    _final_body,
    grid=(GRID,),
    in_specs=[
        pl.BlockSpec((NC, BR, D), lambda i: (0, i, 0)),
        pl.BlockSpec((BR, D), lambda i: (i, 0)),
        pl.BlockSpec((BR, 1), lambda i: (i, 0)),
        pl.BlockSpec((1, D), lambda i: (0, 0)),
    ],
    out_specs=pl.BlockSpec((BR, D), lambda i: (i, 0)),
    out_shape=jax.ShapeDtypeStruct((NP, D), jnp.float32),
)


def kernel(x, edge_index, W1, b1, W2, b2, W3, b3):
    E = edge_index.shape[1]
    pad = EP - E
    pad_idx = PAD_ROW + (jnp.arange(pad, dtype=jnp.int32) % (NP - PAD_ROW))
    src = jnp.concatenate(
        [edge_index[0].astype(jnp.int32), pad_idx]).reshape(NW, S, EB)
    dst = jnp.concatenate(
        [edge_index[1].astype(jnp.int32), pad_idx]).reshape(NW, S, EB)
    x_p = jnp.concatenate([x, jnp.zeros((NP - N, D), jnp.float32)])
    zeros16 = jnp.zeros((NP, 16), jnp.float32)
    ones16 = jnp.ones((EB, 16), jnp.float32)
    zerosND = jnp.zeros((NP, D), jnp.float32)
    b1r = b1.reshape(1, D)
    b2r = b2.reshape(1, D)
    b3r = b3.reshape(1, D)

    deg_parts = _deg_kernel(dst, zeros16, ones16)
    g1, dinv = _first_tc(deg_parts, x_p, W1)
    acc1 = _scatter_kernel(g1, src, dst, zerosND)
    g2 = _mid_tc(acc1, g1, dinv, b1r, W2)
    acc2 = _scatter_kernel(g2, src, dst, zerosND)
    g3 = _mid_tc(acc2, g2, dinv, b2r, W3)
    acc3 = _scatter_kernel(g3, src, dst, zerosND)
    out = _final_tc(acc3, g3, dinv, b3r)
    return out[:N]
